# Initial kernel scaffold; baseline (speedup 1.0000x reference)
import jax
import jax.numpy as jnp
from jax.experimental import pallas as pl


def kernel(x_pool, cluster, s_val):
    raise NotImplementedError("write your pallas kernel here")


# speedup vs baseline: 2.1440x; 2.1440x over previous
import functools
import jax
import jax.numpy as jnp
from jax import lax
from jax.experimental import pallas as pl
from jax.experimental.pallas import tpu as pltpu
from jax.experimental.pallas import tpu_sc as plsc

_N = 100000
_K = 10000
_F = 128
_G = 80
_NG = _N // _G
_NW = 32
_TRIPS = -(-_NG // _NW)

_mesh = plsc.VectorSubcoreMesh(core_axis_name="c", subcore_axis_name="s")


@functools.partial(
    pl.kernel,
    mesh=_mesh,
    out_type=jax.ShapeDtypeStruct((_NG, _G, _F), jnp.float32),
    scratch_types=[
        pltpu.VMEM((_G,), jnp.int32),
        pltpu.VMEM((_G,), jnp.float32),
        pltpu.VMEM((_G, _F), jnp.float32),
        pltpu.SemaphoreType.DMA,
    ],
)
def _lift(x_hbm, idx_hbm, s_hbm, out_hbm, idx_v, s_v, rows_v, sem):
    wid = lax.axis_index("s") * 2 + lax.axis_index("c")

    def group_body(t, carry):
        g = t * _NW + wid

        @pl.when(g < _NG)
        def _():
            pltpu.sync_copy(idx_hbm.at[g], idx_v)
            pltpu.sync_copy(s_hbm.at[g], s_v)
            pltpu.async_copy(x_hbm.at[idx_v], rows_v, sem).wait()

            def blk_body(b, c):
                s16 = s_v[pl.ds(b * 16, 16)]
                for r in range(16):
                    s = s16[r]
                    i = b * 16 + r
                    for j in range(_F // 16):
                        sl = pl.ds(j * 16, 16)
                        rows_v[i, sl] = rows_v[i, sl] * s
                return c

            lax.fori_loop(0, _G // 16, blk_body, 0)
            pltpu.sync_copy(rows_v, out_hbm.at[g])

        return carry

    lax.fori_loop(0, _TRIPS, group_body, 0)


def kernel(x_pool, cluster, s_val):
    idx = cluster.astype(jnp.int32).reshape(_NG, _G)
    s = s_val.reshape(_NG, _G)
    out = _lift(x_pool, idx, s)
    return out.reshape(_N, _F)


# speedup vs baseline: 4.5513x; 2.1228x over previous
import functools
import jax
import jax.numpy as jnp
from jax import lax
from jax.experimental import pallas as pl
from jax.experimental.pallas import tpu as pltpu
from jax.experimental.pallas import tpu_sc as plsc

_N = 100000
_K = 10000
_F = 128
_G = 80
_NG = _N // _G
_NW = 32
_GPW = 40
_NB = 5
_PEEL_LO = 3
_PEEL_HI = 2
_STEADY = _GPW - _PEEL_LO - _PEEL_HI
_OUTER = _STEADY // _NB

_mesh = plsc.VectorSubcoreMesh(core_axis_name="c", subcore_axis_name="s")


@functools.partial(
    pl.kernel,
    mesh=_mesh,
    out_type=jax.ShapeDtypeStruct((_NG, _G, _F), jnp.float32),
    scratch_types=[
        pltpu.VMEM((_GPW * _G,), jnp.int32),
        pltpu.VMEM((_GPW * _G,), jnp.float32),
        pltpu.VMEM((_NB, _G, _F), jnp.float32),
        pltpu.SemaphoreType.DMA,
        pltpu.SemaphoreType.DMA,
        pltpu.SemaphoreType.DMA,
        pltpu.SemaphoreType.DMA,
        pltpu.SemaphoreType.DMA,
        pltpu.SemaphoreType.DMA,
        pltpu.SemaphoreType.DMA,
        pltpu.SemaphoreType.DMA,
        pltpu.SemaphoreType.DMA,
        pltpu.SemaphoreType.DMA,
    ],
)
def _lift(x_hbm, idx_hbm, s_hbm, out_hbm, idx_sl, s_sl, rows_v,
          sg0, sg1, sg2, sg3, sg4, sw0, sw1, sw2, sw3, sw4):
    sems_g = [sg0, sg1, sg2, sg3, sg4]
    sems_w = [sw0, sw1, sw2, sw3, sw4]
    wid = lax.axis_index("s") * 2 + lax.axis_index("c")
    base = jnp.minimum(wid * 39, _NG - _GPW)

    pltpu.sync_copy(idx_hbm.at[pl.ds(base * _G, _GPW * _G)], idx_sl)
    pltpu.sync_copy(s_hbm.at[pl.ds(base * _G, _GPW * _G)], s_sl)

    def fire_gather(g, slot):
        pltpu.async_copy(
            x_hbm.at[idx_sl.at[pl.ds(g * _G, _G)]],
            rows_v.at[slot], sems_g[slot])

    def wait_gather(g, slot):
        pltpu.make_async_copy(
            x_hbm.at[idx_sl.at[pl.ds(g * _G, _G)]],
            rows_v.at[slot], sems_g[slot]).wait()

    def fire_write(g, slot):
        pltpu.async_copy(rows_v.at[slot], out_hbm.at[base + g], sems_w[slot])

    def wait_write(slot):
        pltpu.make_async_copy(
            rows_v.at[slot], out_hbm.at[0], sems_w[slot]).wait()

    def scale(g, slot):
        buf = rows_v.at[slot]

        def blk_body(bb, c):
            s16 = s_sl[pl.ds(g * _G + bb * 16, 16)]
            for r in range(16):
                s = s16[r]
                i = bb * 16 + r
                for j in range(_F // 16):
                    sl = pl.ds(j * 16, 16)
                    buf[i, sl] = buf[i, sl] * s
            return c

        lax.fori_loop(0, _G // 16, blk_body, 0)

    for b in range(_NB):
        fire_gather(b, b)

    for g0 in range(_PEEL_LO):
        wait_gather(g0, g0)
        scale(g0, g0)
        fire_write(g0, g0)

    def outer_body(t0, carry):
        for b in range(_NB):
            g = _PEEL_LO + t0 * _NB + b
            slot = (_PEEL_LO + b) % _NB
            bh = b
            wait_write(bh)
            fire_gather(g + 2, bh)
            wait_gather(g, slot)
            scale(g, slot)
            fire_write(g, slot)
        return carry

    lax.fori_loop(0, _OUTER, outer_body, 0)

    for k in range(_PEEL_HI):
        g = _GPW - _PEEL_HI + k
        slot = g % _NB
        wait_write((g + 2) % _NB)
        wait_gather(g, slot)
        scale(g, slot)
        fire_write(g, slot)

    for slot in range(_PEEL_LO - 1, _NB):
        wait_write(slot)


def kernel(x_pool, cluster, s_val):
    idx = cluster.astype(jnp.int32)
    out = _lift(x_pool, idx, s_val)
    return out.reshape(_N, _F)


# speedup vs baseline: 5.0920x; 1.1188x over previous
import functools
import jax
import jax.numpy as jnp
from jax import lax
from jax.experimental import pallas as pl
from jax.experimental.pallas import tpu as pltpu
from jax.experimental.pallas import tpu_sc as plsc

_N = 100000
_K = 10000
_F = 128
_G = 80
_NG = _N // _G
_NW = 32
_GPW = 40
_NB = 4
_PF = 2
_PEEL_LO = _NB - _PF
_PEEL_HI = _PF
_STEADY = _GPW - _PEEL_LO - _PEEL_HI
_OUTER = _STEADY // _NB

_mesh = plsc.VectorSubcoreMesh(core_axis_name="c", subcore_axis_name="s")


@functools.partial(
    pl.kernel,
    mesh=_mesh,
    out_type=jax.ShapeDtypeStruct((_NG, _G, _F), jnp.float32),
    scratch_types=[
        pltpu.VMEM_SHARED((_K, _F), jnp.float32),
        pltpu.VMEM((_GPW * _G,), jnp.int32),
        pltpu.VMEM((_GPW * _G,), jnp.float32),
        pltpu.VMEM((_NB, _G, _F), jnp.float32),
        pltpu.SemaphoreType.DMA,
        pltpu.SemaphoreType.DMA,
        pltpu.SemaphoreType.DMA,
        pltpu.SemaphoreType.DMA,
        pltpu.SemaphoreType.DMA,
        pltpu.SemaphoreType.DMA,
        pltpu.SemaphoreType.DMA,
        pltpu.SemaphoreType.DMA,
    ],
)
def _lift(x_hbm, idx_hbm, s_hbm, out_hbm, x_sh, idx_sl, s_sl, rows_v,
          sg0, sg1, sg2, sg3, sw0, sw1, sw2, sw3):
    sems_g = [sg0, sg1, sg2, sg3]
    sems_w = [sw0, sw1, sw2, sw3]
    sid = lax.axis_index("s")
    wid = sid * 2 + lax.axis_index("c")
    base = jnp.minimum(wid * _GPW, _NG - _GPW)

    _C = 632
    off = jnp.minimum(sid * (_C // 8), (_K - _C) // 8) * 8
    pltpu.sync_copy(x_hbm.at[pl.ds(off, _C)], x_sh.at[pl.ds(off, _C)])

    pltpu.sync_copy(idx_hbm.at[pl.ds(base * _G, _GPW * _G)], idx_sl)
    pltpu.sync_copy(s_hbm.at[pl.ds(base * _G, _GPW * _G)], s_sl)
    plsc.subcore_barrier()

    def fire_gather(g, slot):
        pltpu.async_copy(
            x_sh.at[idx_sl.at[pl.ds(g * _G, _G)]],
            rows_v.at[slot], sems_g[slot])

    def wait_gather(g, slot):
        pltpu.make_async_copy(
            x_sh.at[idx_sl.at[pl.ds(g * _G, _G)]],
            rows_v.at[slot], sems_g[slot]).wait()

    def fire_write(g, slot):
        pltpu.async_copy(rows_v.at[slot], out_hbm.at[base + g], sems_w[slot])

    def wait_write(slot):
        pltpu.make_async_copy(
            rows_v.at[slot], out_hbm.at[0], sems_w[slot]).wait()

    def scale(g, slot):
        buf = rows_v.at[slot]

        def blk_body(bb, c):
            s16 = s_sl[pl.ds(g * _G + bb * 16, 16)]
            for r in range(16):
                s = s16[r]
                i = bb * 16 + r
                for j in range(_F // 16):
                    sl = pl.ds(j * 16, 16)
                    buf[i, sl] = buf[i, sl] * s
            return c

        lax.fori_loop(0, _G // 16, blk_body, 0)

    for b in range(_NB):
        fire_gather(b, b)

    for g0 in range(_PEEL_LO):
        wait_gather(g0, g0)
        scale(g0, g0)
        fire_write(g0, g0)

    def outer_body(t0, carry):
        for b in range(_NB):
            g = _PEEL_LO + t0 * _NB + b
            slot = (_PEEL_LO + b) % _NB
            bh = b
            wait_write(bh)
            fire_gather(g + _PF, bh)
            wait_gather(g, slot)
            scale(g, slot)
            fire_write(g, slot)
        return carry

    lax.fori_loop(0, _OUTER, outer_body, 0)

    for k in range(_PEEL_HI):
        g = _GPW - _PEEL_HI + k
        slot = g % _NB
        wait_write((g + _PF) % _NB)
        wait_gather(g, slot)
        scale(g, slot)
        fire_write(g, slot)

    for k in range(_PEEL_HI):
        wait_write((_GPW - _PEEL_HI + k) % _NB)


def kernel(x_pool, cluster, s_val):
    idx = cluster.astype(jnp.int32)
    out = _lift(x_pool, idx, s_val)
    return out.reshape(_N, _F)


# speedup vs baseline: 5.3585x; 1.0523x over previous
import functools
import jax
import jax.numpy as jnp
from jax import lax
from jax.experimental import pallas as pl
from jax.experimental.pallas import tpu as pltpu
from jax.experimental.pallas import tpu_sc as plsc

_N = 100000
_K = 10000
_F = 128
_G = 80
_NG = _N // _G
_NW = 32
_GPW = 40
_NB = 4
_PF = 2
_PEEL_LO = _NB - _PF
_PEEL_HI = _PF
_STEADY = _GPW - _PEEL_LO - _PEEL_HI
_OUTER = _STEADY // _NB

_mesh = plsc.VectorSubcoreMesh(core_axis_name="c", subcore_axis_name="s")


@functools.partial(
    pl.kernel,
    mesh=_mesh,
    out_type=jax.ShapeDtypeStruct((_NG, _G, _F), jnp.float32),
    scratch_types=[
        pltpu.VMEM_SHARED((_K, _F), jnp.float32),
        pltpu.VMEM((_GPW * _G,), jnp.int32),
        pltpu.VMEM((_GPW * _G,), jnp.float32),
        pltpu.VMEM((_NB, _G, _F), jnp.float32),
        pltpu.SemaphoreType.DMA,
        pltpu.SemaphoreType.DMA,
        pltpu.SemaphoreType.DMA,
        pltpu.SemaphoreType.DMA,
        pltpu.SemaphoreType.DMA,
        pltpu.SemaphoreType.DMA,
        pltpu.SemaphoreType.DMA,
        pltpu.SemaphoreType.DMA,
        pltpu.SemaphoreType.DMA,
    ],
)
def _lift(x_hbm, idx_hbm, s_hbm, out_hbm, x_sh, idx_sl, s_sl, rows_v,
          sg0, sg1, sg2, sg3, sw0, sw1, sw2, sw3, sst):
    sems_g = [sg0, sg1, sg2, sg3]
    sems_w = [sw0, sw1, sw2, sw3]
    sid = lax.axis_index("s")
    wid = sid * 2 + lax.axis_index("c")
    base = jnp.minimum(wid * _GPW, _NG - _GPW)

    _C = 632
    off = jnp.minimum(sid * (_C // 8), (_K - _C) // 8) * 8
    stage_cp = pltpu.async_copy(
        x_hbm.at[pl.ds(off, _C)], x_sh.at[pl.ds(off, _C)], sst)

    pltpu.sync_copy(idx_hbm.at[pl.ds(base * _G, _GPW * _G)], idx_sl)
    pltpu.sync_copy(s_hbm.at[pl.ds(base * _G, _GPW * _G)], s_sl)

    def fire_gather(g, slot, src):
        pltpu.async_copy(
            src.at[idx_sl.at[pl.ds(g * _G, _G)]],
            rows_v.at[slot], sems_g[slot])

    def wait_gather(g, slot, src):
        pltpu.make_async_copy(
            src.at[idx_sl.at[pl.ds(g * _G, _G)]],
            rows_v.at[slot], sems_g[slot]).wait()

    def fire_write(g, slot):
        pltpu.async_copy(rows_v.at[slot], out_hbm.at[base + g], sems_w[slot])

    def wait_write(slot):
        pltpu.make_async_copy(
            rows_v.at[slot], out_hbm.at[0], sems_w[slot]).wait()

    def scale(g, slot):
        buf = rows_v.at[slot]

        def blk_body(bb, c):
            s16 = s_sl[pl.ds(g * _G + bb * 16, 16)]
            for r in range(16):
                s = s16[r]
                i = bb * 16 + r
                for j in range(_F // 16):
                    sl = pl.ds(j * 16, 16)
                    buf[i, sl] = buf[i, sl] * s
            return c

        lax.fori_loop(0, _G // 16, blk_body, 0)

    for b in range(_NB):
        fire_gather(b, b, x_hbm)

    for g0 in range(_PEEL_LO):
        wait_gather(g0, g0, x_hbm)
        scale(g0, g0)
        fire_write(g0, g0)

    stage_cp.wait()
    plsc.subcore_barrier()

    def outer_body(t0, carry):
        for b in range(_NB):
            g = _PEEL_LO + t0 * _NB + b
            slot = (_PEEL_LO + b) % _NB
            bh = b
            wait_write(bh)
            fire_gather(g + _PF, bh, x_sh)
            wait_gather(g, slot, x_sh)
            scale(g, slot)
            fire_write(g, slot)
        return carry

    lax.fori_loop(0, _OUTER, outer_body, 0)

    for k in range(_PEEL_HI):
        g = _GPW - _PEEL_HI + k
        slot = g % _NB
        wait_write((g + _PF) % _NB)
        wait_gather(g, slot, x_sh)
        scale(g, slot)
        fire_write(g, slot)

    for k in range(_PEEL_HI):
        wait_write((_GPW - _PEEL_HI + k) % _NB)


def kernel(x_pool, cluster, s_val):
    idx = cluster.astype(jnp.int32)
    out = _lift(x_pool, idx, s_val)
    return out.reshape(_N, _F)
